# baseline (speedup 1.0000x reference)
import functools

import jax
import jax.numpy as jnp
from jax import lax
from jax.experimental import pallas as pl
from jax.experimental.pallas import tpu as pltpu
from jax.experimental.pallas import tpu_sc as plsc

_BS = 2
_NB = 4
_CHUNK = 128


def _build(B, V, D, NC, NS):
    NW = NC * NS
    b_per_w = B // NW
    n_blocks = b_per_w // _BS
    n_chunks = b_per_w // _CHUNK
    blocks_per_chunk = _CHUNK // _BS
    iters = n_blocks // _NB
    ipc = blocks_per_chunk // _NB
    mesh = plsc.VectorSubcoreMesh(core_axis_name="c", subcore_axis_name="s")

    @functools.partial(
        pl.kernel,
        mesh=mesh,
        out_type=jax.ShapeDtypeStruct((B, 128), jnp.float32),
        scratch_types=[
            pltpu.VMEM((b_per_w,), jnp.int32),
            pltpu.VMEM((n_chunks, _CHUNK), jnp.int32),
            pltpu.VMEM((_NB * _BS, D, 128), jnp.float32),
            pltpu.VMEM((_CHUNK, 128), jnp.float32),
            pltpu.SemaphoreType.DMA,
            pltpu.SemaphoreType.DMA,
            pltpu.SemaphoreType.DMA,
            pltpu.SemaphoreType.DMA,
            pltpu.SemaphoreType.DMA,
        ],
        compiler_params=pltpu.CompilerParams(needs_layout_passes=False),
    )
    def gather_kernel(tab_hbm, sr_hbm, perm_hbm, out_hbm, idx_v, perm_v,
                      slots, rows, sem0, sem1, sem2, sem3, sem_sc):
        wid = lax.axis_index("s") * NC + lax.axis_index("c")
        base = wid * b_per_w
        pltpu.sync_copy(sr_hbm.at[pl.ds(base, b_per_w)], idx_v)
        pltpu.sync_copy(perm_hbm.at[pl.ds(wid * n_chunks, n_chunks)], perm_v)
        lanes = lax.iota(jnp.int32, 16)
        sems = [sem0, sem1, sem2, sem3]
        dummy = tab_hbm.at[:, pl.ds(0, 128)]

        def plan(b, bank, carry):
            ps, pnf, pslot = carry
            off = pl.multiple_of((b // 8) * 16, 16)
            ivec = idx_v[pl.ds(off, 16)]
            lo = (b % 8) * 2
            rs = [jnp.sum(jnp.where(lanes == lo + l, ivec, 0))
                  for l in range(_BS)]
            ss = [lax.bitwise_and(r, -128) for r in rs]
            f0 = jnp.logical_or(ss[0] != ps, pnf == 0)
            nf0 = jnp.where(f0, 1, 0)
            slot0 = jnp.where(f0, bank * _BS, pslot)
            f1 = ss[1] != ss[0]
            nf1 = nf0 + jnp.where(f1, 1, 0)
            slot1 = jnp.where(f1, bank * _BS + nf0, slot0)
            return dict(rs=rs, ss=ss, f=[f0, f1], nf=[nf0, nf1],
                        slot=[slot0, slot1], carry=(ss[1], nf1, slot1))

        def issue(p, bank, sem):
            for l in range(_BS):
                @pl.when(p["f"][l])
                def _(l=l):
                    s = pl.multiple_of(p["ss"][l], 128)
                    pltpu.async_copy(tab_hbm.at[:, pl.ds(s, 128)],
                                     slots.at[p["slot"][l]], sem)

        def drain(p, sem):
            for k in range(_BS):
                @pl.when(p["nf"][1] > k)
                def _(k=k):
                    pltpu.make_async_copy(dummy, slots.at[k], sem).wait()

        def extract(b, p):
            for l in range(_BS):
                col = jnp.full((16,), lax.bitwise_and(p["rs"][l], 127),
                               jnp.int32)
                row = (b % blocks_per_chunk) * _BS + l
                for k in range(D // 16):
                    vals = plsc.load_gather(
                        slots.at[p["slot"][l]], [lanes + 16 * k, col]
                    )
                    rows[row, pl.ds(16 * k, 16)] = vals

        carry0 = (jnp.int32(-1), jnp.int32(1), jnp.int32(0))
        issue(plan(0, 0, carry0), 0, sem0)

        def body(i, carry):
            b0 = _NB * i

            @pl.when(jnp.logical_and(i % ipc == 0, i > 0))
            def _():
                pltpu.make_async_copy(out_hbm.at[pl.ds(0, _CHUNK)],
                                      rows, sem_sc).wait()

            plans = []
            c = carry
            for j in range(_NB):
                p = plan(b0 + j, j, c)
                plans.append(p)
                c = p["carry"]

            for j in range(_NB):
                if j + 1 < _NB:
                    issue(plans[j + 1], j + 1, sems[j + 1])
                else:
                    p_next = plan(b0 + _NB, 0, c)

                    @pl.when(b0 + _NB < n_blocks)
                    def _():
                        issue(p_next, 0, sem0)
                drain(plans[j], sems[j])
                extract(b0 + j, plans[j])

            @pl.when(i % ipc == ipc - 1)
            def _():
                pltpu.async_copy(rows, out_hbm.at[perm_v.at[i // ipc]],
                                 sem_sc)
            return c

        lax.fori_loop(0, iters, body, carry0)
        pltpu.make_async_copy(out_hbm.at[pl.ds(0, _CHUNK)],
                              rows, sem_sc).wait()

    return gather_kernel


def kernel(cls, embedding):
    (B,) = cls.shape
    V, D = embedding.shape
    info = plsc.get_sparse_core_info()
    NC, NS = info.num_cores, info.num_subcores
    sr, perm = lax.sort_key_val(cls, lax.iota(jnp.int32, B))
    out128 = _build(B, V, D, NC, NS)(embedding.T, sr,
                                     perm.reshape(B // 128, 128))
    return out128[:, :D]


# speedup vs baseline: 1.2993x; 1.2993x over previous
import functools

import jax
import jax.numpy as jnp
from jax import lax
from jax.experimental import pallas as pl
from jax.experimental.pallas import tpu as pltpu
from jax.experimental.pallas import tpu_sc as plsc

_BS = 4
_NBANK = 3
_CHUNK = 128


def _build(B, V, D, NC, NS):
    NW = NC * NS
    b_per_w = B // NW
    n_blocks = b_per_w // _BS
    n_chunks = b_per_w // _CHUNK
    blocks_per_chunk = _CHUNK // _BS
    mesh = plsc.VectorSubcoreMesh(core_axis_name="c", subcore_axis_name="s")

    @functools.partial(
        pl.kernel,
        mesh=mesh,
        out_type=jax.ShapeDtypeStruct((B, 128), jnp.float32),
        scratch_types=[
            pltpu.VMEM((b_per_w,), jnp.int32),
            pltpu.VMEM((n_chunks, _CHUNK), jnp.int32),
            pltpu.VMEM((_NBANK * _BS, D, 128), jnp.float32),
            pltpu.VMEM((_CHUNK, 128), jnp.float32),
            pltpu.SemaphoreType.DMA,
            pltpu.SemaphoreType.DMA,
            pltpu.SemaphoreType.DMA,
            pltpu.SemaphoreType.DMA,
        ],
        compiler_params=pltpu.CompilerParams(needs_layout_passes=False),
    )
    def gather_kernel(tab_hbm, sr_hbm, perm_hbm, out_hbm, idx_v, perm_v,
                      slots, rows, sem0, sem1, sem2, sem_sc):
        wid = lax.axis_index("s") * NC + lax.axis_index("c")
        base = wid * b_per_w
        pltpu.sync_copy(sr_hbm.at[pl.ds(base, b_per_w)], idx_v)
        pltpu.sync_copy(perm_hbm.at[pl.ds(wid * n_chunks, n_chunks)], perm_v)
        lanes = lax.iota(jnp.int32, 16)
        sems = [sem0, sem1, sem2]
        dummy = tab_hbm.at[:, pl.ds(0, 128)]

        def plan(b, carry):
            ps, pnf, pslot = carry
            off = pl.multiple_of((b // 4) * 16, 16)
            ivec = idx_v[pl.ds(off, 16)]
            lo = (b % 4) * 4
            rs = [jnp.sum(jnp.where(lanes == lo + l, ivec, 0))
                  for l in range(_BS)]
            ss = [lax.bitwise_and(r, -128) for r in rs]
            sbase = (b % _NBANK) * _BS
            f = [jnp.logical_or(ss[0] != ps, pnf == 0)]
            nf = [jnp.where(f[0], 1, 0)]
            slot = [jnp.where(f[0], sbase, pslot)]
            for l in range(1, _BS):
                fl = ss[l] != ss[l - 1]
                f.append(fl)
                slot.append(jnp.where(fl, sbase + nf[l - 1], slot[l - 1]))
                nf.append(nf[l - 1] + jnp.where(fl, 1, 0))
            return dict(rs=rs, ss=ss, f=f, nf=nf, slot=slot,
                        sbase=sbase, carry=(ss[-1], nf[-1], slot[-1]))

        def issue(p, sem):
            for l in range(_BS):
                @pl.when(p["f"][l])
                def _(l=l):
                    s = pl.multiple_of(p["ss"][l], 128)
                    pltpu.async_copy(tab_hbm.at[:, pl.ds(s, 128)],
                                     slots.at[p["slot"][l]], sem)

        def drain(p, sem):
            for k in range(_BS):
                @pl.when(p["nf"][_BS - 1] > k)
                def _(k=k):
                    pltpu.make_async_copy(dummy, slots.at[k], sem).wait()

        def extract(b, p):
            for l in range(_BS):
                col = jnp.full((16,), lax.bitwise_and(p["rs"][l], 127),
                               jnp.int32)
                row = (b % blocks_per_chunk) * _BS + l
                for k in range(D // 16):
                    vals = plsc.load_gather(
                        slots.at[p["slot"][l]], [lanes + 16 * k, col]
                    )
                    rows[row, pl.ds(16 * k, 16)] = vals

        carry0 = (jnp.int32(-1), jnp.int32(1), jnp.int32(0))
        issue(plan(0, carry0), sem0)

        def body(b, carry):
            p = plan(b, carry)
            p_next = plan(b + 1, p["carry"])
            for k in range(_NBANK):
                @pl.when(jnp.logical_and((b + 1) % _NBANK == k,
                                         b + 1 < n_blocks))
                def _(k=k):
                    issue(p_next, sems[k])
            for k in range(_NBANK):
                @pl.when(b % _NBANK == k)
                def _(k=k):
                    drain(p, sems[k])

            @pl.when(jnp.logical_and(b % blocks_per_chunk == 0, b > 0))
            def _():
                pltpu.make_async_copy(out_hbm.at[pl.ds(0, _CHUNK)],
                                      rows, sem_sc).wait()

            extract(b, p)

            @pl.when(b % blocks_per_chunk == blocks_per_chunk - 1)
            def _():
                pltpu.async_copy(rows,
                                 out_hbm.at[perm_v.at[b // blocks_per_chunk]],
                                 sem_sc)
            return p["carry"]

        lax.fori_loop(0, n_blocks, body, carry0)
        pltpu.make_async_copy(out_hbm.at[pl.ds(0, _CHUNK)],
                              rows, sem_sc).wait()

    return gather_kernel


def kernel(cls, embedding):
    (B,) = cls.shape
    V, D = embedding.shape
    info = plsc.get_sparse_core_info()
    NC, NS = info.num_cores, info.num_subcores
    sr, perm = lax.sort_key_val(cls, lax.iota(jnp.int32, B))
    out128 = _build(B, V, D, NC, NS)(embedding.T, sr,
                                     perm.reshape(B // 128, 128))
    return out128[:, :D]


# speedup vs baseline: 1.3033x; 1.0031x over previous
import functools

import jax
import jax.numpy as jnp
from jax import lax
from jax.experimental import pallas as pl
from jax.experimental.pallas import tpu as pltpu
from jax.experimental.pallas import tpu_sc as plsc

_BS = 4
_NBANK = 3
_CHUNK = 128


def _build(B, V, D, NC, NS):
    NW = NC * NS
    b_per_w = B // NW
    n_blocks = b_per_w // _BS
    n_chunks = b_per_w // _CHUNK
    blocks_per_chunk = _CHUNK // _BS
    mesh = plsc.VectorSubcoreMesh(core_axis_name="c", subcore_axis_name="s")

    @functools.partial(
        pl.kernel,
        mesh=mesh,
        out_type=jax.ShapeDtypeStruct((B, 128), jnp.float32),
        scratch_types=[
            pltpu.VMEM((b_per_w + 16,), jnp.int32),
            pltpu.VMEM((n_chunks, _CHUNK), jnp.int32),
            pltpu.VMEM((_NBANK * _BS, D, 128), jnp.float32),
            pltpu.VMEM((_CHUNK, 128), jnp.float32),
            pltpu.SemaphoreType.DMA,
            pltpu.SemaphoreType.DMA,
            pltpu.SemaphoreType.DMA,
            pltpu.SemaphoreType.DMA,
        ],
        compiler_params=pltpu.CompilerParams(needs_layout_passes=False),
    )
    def gather_kernel(tab_hbm, sr_hbm, perm_hbm, out_hbm, idx_v, perm_v,
                      slots, rows, sem0, sem1, sem2, sem_sc):
        wid = lax.axis_index("s") * NC + lax.axis_index("c")
        base = wid * b_per_w
        pltpu.sync_copy(sr_hbm.at[pl.ds(base, b_per_w)],
                        idx_v.at[pl.ds(0, b_per_w)])
        pltpu.sync_copy(perm_hbm.at[pl.ds(wid * n_chunks, n_chunks)], perm_v)
        lanes = lax.iota(jnp.int32, 16)
        sems = [sem0, sem1, sem2]
        dummy = tab_hbm.at[:, pl.ds(0, 128)]

        def scan_rs(b):
            off = pl.multiple_of((b // 4) * 16, 16)
            ivec = idx_v[pl.ds(off, 16)]
            lo = (b % 4) * 4
            return [jnp.sum(jnp.where(lanes == lo + l, ivec, 0))
                    for l in range(_BS)]

        def plan(b, rs, carry):
            ps, pnf, pslot = carry
            ss = [lax.bitwise_and(r, -128) for r in rs]
            sbase = (b % _NBANK) * _BS
            f = [jnp.logical_or(ss[0] != ps, pnf == 0)]
            nf = [jnp.where(f[0], 1, 0)]
            slot = [jnp.where(f[0], sbase, pslot)]
            for l in range(1, _BS):
                fl = ss[l] != ss[l - 1]
                f.append(fl)
                slot.append(jnp.where(fl, sbase + nf[l - 1], slot[l - 1]))
                nf.append(nf[l - 1] + jnp.where(fl, 1, 0))
            return dict(rs=rs, ss=ss, f=f, nf=nf, slot=slot,
                        sbase=sbase, carry=(ss[-1], nf[-1], slot[-1]))

        def issue(p, sem):
            for l in range(_BS):
                @pl.when(p["f"][l])
                def _(l=l):
                    s = pl.multiple_of(p["ss"][l], 128)
                    pltpu.async_copy(tab_hbm.at[:, pl.ds(s, 128)],
                                     slots.at[p["slot"][l]], sem)

        def drain(p, sem):
            for k in range(_BS):
                @pl.when(p["nf"][_BS - 1] > k)
                def _(k=k):
                    pltpu.make_async_copy(dummy, slots.at[k], sem).wait()

        def extract(b, p):
            for l in range(_BS):
                col = jnp.full((16,), lax.bitwise_and(p["rs"][l], 127),
                               jnp.int32)
                row = (b % blocks_per_chunk) * _BS + l
                for k in range(D // 16):
                    vals = plsc.load_gather(
                        slots.at[p["slot"][l]], [lanes + 16 * k, col]
                    )
                    rows[row, pl.ds(16 * k, 16)] = vals

        chain0 = (jnp.int32(-1), jnp.int32(1), jnp.int32(0))
        rs0 = scan_rs(0)
        issue(plan(0, rs0, chain0), sem0)
        carry0 = chain0 + tuple(rs0)

        def body(b, carry):
            chain, rs_b = carry[:3], list(carry[3:])
            p = plan(b, rs_b, chain)
            rs_next = scan_rs(b + 1)
            p_next = plan(b + 1, rs_next, p["carry"])
            for k in range(_NBANK):
                @pl.when(jnp.logical_and((b + 1) % _NBANK == k,
                                         b + 1 < n_blocks))
                def _(k=k):
                    issue(p_next, sems[k])
            for k in range(_NBANK):
                @pl.when(b % _NBANK == k)
                def _(k=k):
                    drain(p, sems[k])

            @pl.when(jnp.logical_and(b % blocks_per_chunk == 0, b > 0))
            def _():
                pltpu.make_async_copy(out_hbm.at[pl.ds(0, _CHUNK)],
                                      rows, sem_sc).wait()

            extract(b, p)

            @pl.when(b % blocks_per_chunk == blocks_per_chunk - 1)
            def _():
                pltpu.async_copy(rows,
                                 out_hbm.at[perm_v.at[b // blocks_per_chunk]],
                                 sem_sc)
            return p["carry"] + tuple(rs_next)

        lax.fori_loop(0, n_blocks, body, carry0)
        pltpu.make_async_copy(out_hbm.at[pl.ds(0, _CHUNK)],
                              rows, sem_sc).wait()

    return gather_kernel


def kernel(cls, embedding):
    (B,) = cls.shape
    V, D = embedding.shape
    info = plsc.get_sparse_core_info()
    NC, NS = info.num_cores, info.num_subcores
    sr, perm = lax.sort_key_val(cls, lax.iota(jnp.int32, B))
    out128 = _build(B, V, D, NC, NS)(embedding.T, sr,
                                     perm.reshape(B // 128, 128))
    return out128[:, :D]
